# initial kernel scaffold (unmeasured)
import jax
import jax.numpy as jnp
from jax import lax
from jax.experimental import pallas as pl
from jax.experimental.pallas import tpu as pltpu


def kernel(
    x,
):
    def body(*refs):
        pass

    out_shape = jax.ShapeDtypeStruct(..., jnp.float32)
    return pl.pallas_call(body, out_shape=out_shape)(...)


# baseline (device time: 81028 ns/iter reference)
import jax
import jax.numpy as jnp
from jax import lax
from jax.experimental import pallas as pl
from jax.experimental.pallas import tpu as pltpu

N_DEV = 4


def kernel(x):
    m_per, n = x.shape

    def body(x_ref, out_ref, comm_ref, send_sems, recv_sems):
        my_pos = lax.axis_index("i")
        left = (my_pos - 1) % N_DEV
        right = (my_pos + 1) % N_DEV

        barrier_sem = pltpu.get_barrier_semaphore()
        for nbr in [left, right]:
            pl.semaphore_signal(
                barrier_sem, inc=1,
                device_id=(nbr,), device_id_type=pl.DeviceIdType.MESH,
            )
        pl.semaphore_wait(barrier_sem, 2)

        out_ref[pl.ds(my_pos * m_per, m_per), :] = x_ref[:, :]
        comm_ref[0, :, :] = x_ref[:, :]

        for h in range(N_DEV - 1):
            send_slot = h % 2
            recv_slot = (h + 1) % 2
            rdma = pltpu.make_async_remote_copy(
                src_ref=comm_ref.at[send_slot],
                dst_ref=comm_ref.at[recv_slot],
                send_sem=send_sems.at[send_slot],
                recv_sem=recv_sems.at[recv_slot],
                device_id=(right,),
                device_id_type=pl.DeviceIdType.MESH,
            )
            rdma.start()
            rdma.wait()

            origin = (my_pos - h - 1) % N_DEV
            out_ref[pl.ds(origin * m_per, m_per), :] = comm_ref[recv_slot, :, :]

    return pl.pallas_call(
        body,
        out_shape=jax.ShapeDtypeStruct((N_DEV * m_per, n), x.dtype),
        in_specs=[pl.BlockSpec(memory_space=pltpu.VMEM)],
        out_specs=pl.BlockSpec(memory_space=pltpu.VMEM),
        scratch_shapes=[
            pltpu.VMEM((2, m_per, n), x.dtype),
            pltpu.SemaphoreType.DMA((2,)),
            pltpu.SemaphoreType.DMA((2,)),
        ],
        compiler_params=pltpu.CompilerParams(collective_id=0),
    )(x)


# device time: 46812 ns/iter; 1.7309x vs baseline; 1.7309x over previous
import jax
import jax.numpy as jnp
from jax import lax
from jax.experimental import pallas as pl
from jax.experimental.pallas import tpu as pltpu

N_DEV = 4
N_HOPS = N_DEV - 1


def kernel(x):
    m_per, n = x.shape
    m_half = m_per // 2

    def body(x_ref, out_ref, cw_ref, ccw_ref,
             send_cw, recv_cw, send_ccw, recv_ccw):
        my_pos = lax.axis_index("i")
        left = (my_pos - 1) % N_DEV
        right = (my_pos + 1) % N_DEV

        barrier_sem = pltpu.get_barrier_semaphore()
        for nbr in [left, right]:
            pl.semaphore_signal(
                barrier_sem, inc=1,
                device_id=(nbr,), device_id_type=pl.DeviceIdType.MESH,
            )
        pl.semaphore_wait(barrier_sem, 2)

        rdmas = []
        for h in range(N_HOPS):
            src_cw = x_ref.at[pl.ds(0, m_half), :] if h == 0 else cw_ref.at[h - 1]
            src_ccw = (x_ref.at[pl.ds(m_half, m_half), :] if h == 0
                       else ccw_ref.at[h - 1])
            rdma_cw = pltpu.make_async_remote_copy(
                src_ref=src_cw,
                dst_ref=cw_ref.at[h],
                send_sem=send_cw.at[h],
                recv_sem=recv_cw.at[h],
                device_id=(right,),
                device_id_type=pl.DeviceIdType.MESH,
            )
            rdma_ccw = pltpu.make_async_remote_copy(
                src_ref=src_ccw,
                dst_ref=ccw_ref.at[h],
                send_sem=send_ccw.at[h],
                recv_sem=recv_ccw.at[h],
                device_id=(left,),
                device_id_type=pl.DeviceIdType.MESH,
            )
            rdma_cw.start()
            rdma_ccw.start()
            rdmas.append((rdma_cw, rdma_ccw))

            if h == 0:
                out_ref[pl.ds(my_pos * m_per, m_per), :] = x_ref[:, :]
            else:
                o_cw = (my_pos - h) % N_DEV
                o_ccw = (my_pos + h) % N_DEV
                out_ref[pl.ds(o_cw * m_per, m_half), :] = cw_ref[h - 1]
                out_ref[pl.ds(o_ccw * m_per + m_half, m_half), :] = ccw_ref[h - 1]

            rdma_cw.wait_recv()
            rdma_ccw.wait_recv()

        o_cw = (my_pos - N_HOPS) % N_DEV
        o_ccw = (my_pos + N_HOPS) % N_DEV
        out_ref[pl.ds(o_cw * m_per, m_half), :] = cw_ref[N_HOPS - 1]
        out_ref[pl.ds(o_ccw * m_per + m_half, m_half), :] = ccw_ref[N_HOPS - 1]

        for rdma_cw, rdma_ccw in rdmas:
            rdma_cw.wait_send()
            rdma_ccw.wait_send()

    return pl.pallas_call(
        body,
        out_shape=jax.ShapeDtypeStruct((N_DEV * m_per, n), x.dtype),
        in_specs=[pl.BlockSpec(memory_space=pltpu.VMEM)],
        out_specs=pl.BlockSpec(memory_space=pltpu.VMEM),
        scratch_shapes=[
            pltpu.VMEM((N_HOPS, m_half, n), x.dtype),
            pltpu.VMEM((N_HOPS, m_half, n), x.dtype),
            pltpu.SemaphoreType.DMA((N_HOPS,)),
            pltpu.SemaphoreType.DMA((N_HOPS,)),
            pltpu.SemaphoreType.DMA((N_HOPS,)),
            pltpu.SemaphoreType.DMA((N_HOPS,)),
        ],
        compiler_params=pltpu.CompilerParams(collective_id=0),
    )(x)


# device time: 42498 ns/iter; 1.9066x vs baseline; 1.1015x over previous
import jax
import jax.numpy as jnp
from jax import lax
from jax.experimental import pallas as pl
from jax.experimental.pallas import tpu as pltpu

N_DEV = 4
N_HOPS = N_DEV - 1
N_SEG = 2


def kernel(x):
    m_per, n = x.shape
    m_half = m_per // 2
    m_seg = m_half // N_SEG

    def body(x_ref, out_ref, cw_ref, ccw_ref,
             send_cw, recv_cw, send_ccw, recv_ccw):
        my_pos = lax.axis_index("i")
        left = (my_pos - 1) % N_DEV
        right = (my_pos + 1) % N_DEV

        barrier_sem = pltpu.get_barrier_semaphore()
        for nbr in [left, right]:
            pl.semaphore_signal(
                barrier_sem, inc=1,
                device_id=(nbr,), device_id_type=pl.DeviceIdType.MESH,
            )
        pl.semaphore_wait(barrier_sem, 2)

        def make_pair(h, s):
            if h == 0:
                src_cw = x_ref.at[pl.ds(s * m_seg, m_seg), :]
                src_ccw = x_ref.at[pl.ds(m_half + s * m_seg, m_seg), :]
            else:
                src_cw = cw_ref.at[h - 1, pl.ds(s * m_seg, m_seg), :]
                src_ccw = ccw_ref.at[h - 1, pl.ds(s * m_seg, m_seg), :]
            rdma_cw = pltpu.make_async_remote_copy(
                src_ref=src_cw,
                dst_ref=cw_ref.at[h, pl.ds(s * m_seg, m_seg), :],
                send_sem=send_cw.at[h, s],
                recv_sem=recv_cw.at[h, s],
                device_id=(right,),
                device_id_type=pl.DeviceIdType.MESH,
            )
            rdma_ccw = pltpu.make_async_remote_copy(
                src_ref=src_ccw,
                dst_ref=ccw_ref.at[h, pl.ds(s * m_seg, m_seg), :],
                send_sem=send_ccw.at[h, s],
                recv_sem=recv_ccw.at[h, s],
                device_id=(left,),
                device_id_type=pl.DeviceIdType.MESH,
            )
            return rdma_cw, rdma_ccw

        def store_hop(h):
            o_cw = (my_pos - h - 1) % N_DEV
            o_ccw = (my_pos + h + 1) % N_DEV
            out_ref[pl.ds(o_cw * m_per, m_half), :] = cw_ref[h]
            out_ref[pl.ds(o_ccw * m_per + m_half, m_half), :] = ccw_ref[h]

        rdmas = {}
        for s in range(N_SEG):
            pair = make_pair(0, s)
            pair[0].start()
            pair[1].start()
            rdmas[(0, s)] = pair

        out_ref[pl.ds(my_pos * m_per, m_per), :] = x_ref[:, :]

        for h in range(1, N_HOPS):
            for s in range(N_SEG):
                rdmas[(h - 1, s)][0].wait_recv()
                rdmas[(h - 1, s)][1].wait_recv()
                pair = make_pair(h, s)
                pair[0].start()
                pair[1].start()
                rdmas[(h, s)] = pair
            store_hop(h - 1)

        for s in range(N_SEG):
            rdmas[(N_HOPS - 1, s)][0].wait_recv()
            rdmas[(N_HOPS - 1, s)][1].wait_recv()
        store_hop(N_HOPS - 1)

        for pair in rdmas.values():
            pair[0].wait_send()
            pair[1].wait_send()

    return pl.pallas_call(
        body,
        out_shape=jax.ShapeDtypeStruct((N_DEV * m_per, n), x.dtype),
        in_specs=[pl.BlockSpec(memory_space=pltpu.VMEM)],
        out_specs=pl.BlockSpec(memory_space=pltpu.VMEM),
        scratch_shapes=[
            pltpu.VMEM((N_HOPS, m_half, n), x.dtype),
            pltpu.VMEM((N_HOPS, m_half, n), x.dtype),
            pltpu.SemaphoreType.DMA((N_HOPS, N_SEG)),
            pltpu.SemaphoreType.DMA((N_HOPS, N_SEG)),
            pltpu.SemaphoreType.DMA((N_HOPS, N_SEG)),
            pltpu.SemaphoreType.DMA((N_HOPS, N_SEG)),
        ],
        compiler_params=pltpu.CompilerParams(collective_id=0),
    )(x)
